# Initial kernel scaffold; baseline (speedup 1.0000x reference)
import functools

import jax
import jax.numpy as jnp
from jax import lax
from jax.experimental import pallas as pl
from jax.experimental.pallas import tpu as pltpu
from jax.experimental.pallas import tpu_sc as plsc

BATCH = 16384
NUM_FIELDS = 26
EMBED_DIM = 32
TOTAL = BATCH * NUM_FIELDS
NC = 2
NS = 16
NW = NC * NS
B_PER_W = TOTAL // NW
CHUNK = 1664
N_CHUNKS = B_PER_W // CHUNK

_mesh = plsc.VectorSubcoreMesh(core_axis_name="c", subcore_axis_name="s")


@functools.partial(
    pl.kernel,
    out_type=jax.ShapeDtypeStruct((TOTAL, EMBED_DIM), jnp.float32),
    mesh=_mesh,
    scratch_types=[
        pltpu.VMEM((B_PER_W,), jnp.int32),
        pltpu.VMEM((CHUNK, EMBED_DIM), jnp.float32),
        pltpu.SemaphoreType.DMA,
    ],
)
def _embed_gather(idx_hbm, table_hbm, out_hbm, idx_v, rows_v, sem):
    wid = lax.axis_index("s") * NC + lax.axis_index("c")
    base = wid * B_PER_W
    pltpu.sync_copy(idx_hbm.at[pl.ds(base, B_PER_W)], idx_v)
    for c in range(N_CHUNKS):
        off = c * CHUNK
        pltpu.async_copy(
            table_hbm.at[idx_v.at[pl.ds(off, CHUNK)]], rows_v, sem
        ).wait()
        pltpu.sync_copy(rows_v, out_hbm.at[pl.ds(base + off, CHUNK)])


def kernel(x, table):
    flat = x.reshape(TOTAL).astype(jnp.int32)
    out = _embed_gather(flat, table)
    return out.reshape(BATCH, NUM_FIELDS, EMBED_DIM)


# speedup vs baseline: 1.5690x; 1.5690x over previous
import functools

import jax
import jax.numpy as jnp
from jax import lax
from jax.experimental import pallas as pl
from jax.experimental.pallas import tpu as pltpu
from jax.experimental.pallas import tpu_sc as plsc

BATCH = 16384
NUM_FIELDS = 26
EMBED_DIM = 32
TOTAL = BATCH * NUM_FIELDS
NC = 2
NS = 16
NW = NC * NS
B_PER_W = TOTAL // NW
CHUNK = 1664
N_CHUNKS = B_PER_W // CHUNK

_mesh = plsc.VectorSubcoreMesh(core_axis_name="c", subcore_axis_name="s")


@functools.partial(
    pl.kernel,
    out_type=jax.ShapeDtypeStruct((TOTAL, EMBED_DIM), jnp.float32),
    mesh=_mesh,
    scratch_types=[
        pltpu.VMEM((B_PER_W,), jnp.int32),
        pltpu.VMEM((CHUNK, EMBED_DIM), jnp.float32),
        pltpu.SemaphoreType.DMA,
    ],
    compiler_params=pltpu.CompilerParams(use_tc_tiling_on_sc=False),
)
def _embed_gather(idx_hbm, table_hbm, out_hbm, idx_v, rows_v, sem):
    wid = lax.axis_index("s") * NC + lax.axis_index("c")
    base = wid * B_PER_W
    pltpu.sync_copy(idx_hbm.at[pl.ds(base, B_PER_W)], idx_v)
    for c in range(N_CHUNKS):
        off = c * CHUNK
        pltpu.async_copy(
            table_hbm.at[idx_v.at[pl.ds(off, CHUNK)]], rows_v, sem
        ).wait()
        pltpu.sync_copy(rows_v, out_hbm.at[pl.ds(base + off, CHUNK)])


def kernel(x, table):
    flat = x.reshape(TOTAL).astype(jnp.int32)
    out = _embed_gather(flat, table)
    return out.reshape(BATCH, NUM_FIELDS, EMBED_DIM)


# speedup vs baseline: 1.5761x; 1.0045x over previous
import functools

import jax
import jax.numpy as jnp
from jax import lax
from jax.experimental import pallas as pl
from jax.experimental.pallas import tpu as pltpu
from jax.experimental.pallas import tpu_sc as plsc

BATCH = 16384
NUM_FIELDS = 26
EMBED_DIM = 32
TOTAL = BATCH * NUM_FIELDS
NC = 2
NS = 16
NW = NC * NS
B_PER_W = TOTAL // NW
CHUNK = 832
N_CHUNKS = B_PER_W // CHUNK
NBUF = 4

_mesh = plsc.VectorSubcoreMesh(core_axis_name="c", subcore_axis_name="s")


@functools.partial(
    pl.kernel,
    out_type=jax.ShapeDtypeStruct((TOTAL, EMBED_DIM), jnp.float32),
    mesh=_mesh,
    scratch_types=[
        pltpu.VMEM((B_PER_W,), jnp.int32),
        pltpu.VMEM((NBUF, CHUNK, EMBED_DIM), jnp.float32),
        [pltpu.SemaphoreType.DMA] * NBUF,
        [pltpu.SemaphoreType.DMA] * NBUF,
    ],
    compiler_params=pltpu.CompilerParams(use_tc_tiling_on_sc=False),
)
def _embed_gather(idx_hbm, table_hbm, out_hbm, idx_v, rows_v, gsems, ssems):
    wid = lax.axis_index("s") * NC + lax.axis_index("c")
    base = wid * B_PER_W
    pltpu.sync_copy(idx_hbm.at[pl.ds(base, B_PER_W)], idx_v)

    def start_gather(c):
        b = c % NBUF
        return pltpu.async_copy(
            table_hbm.at[idx_v.at[pl.ds(c * CHUNK, CHUNK)]],
            rows_v.at[b],
            gsems[b],
        )

    def start_store(c):
        b = c % NBUF
        return pltpu.async_copy(
            rows_v.at[b], out_hbm.at[pl.ds(base + c * CHUNK, CHUNK)], ssems[b]
        )

    gathers = [None] * N_CHUNKS
    stores = [None] * N_CHUNKS
    for c in range(min(NBUF - 1, N_CHUNKS)):
        gathers[c] = start_gather(c)
    for c in range(N_CHUNKS):
        if c > 0:
            stores[c - 1].wait()
        g = c + NBUF - 1
        if g < N_CHUNKS:
            gathers[g] = start_gather(g)
        gathers[c].wait()
        stores[c] = start_store(c)
    stores[N_CHUNKS - 1].wait()


def kernel(x, table):
    flat = x.reshape(TOTAL).astype(jnp.int32)
    out = _embed_gather(flat, table)
    return out.reshape(BATCH, NUM_FIELDS, EMBED_DIM)


# speedup vs baseline: 1.6004x; 1.0154x over previous
import functools

import jax
import jax.numpy as jnp
from jax import lax
from jax.experimental import pallas as pl
from jax.experimental.pallas import tpu as pltpu
from jax.experimental.pallas import tpu_sc as plsc

BATCH = 16384
NUM_FIELDS = 26
EMBED_DIM = 32
VOCAB_ROWS = 1000000
TOTAL = BATCH * NUM_FIELDS
NC = 2
NS = 16
NW = NC * NS
B_PER_W = TOTAL // NW
CHUNK = 832
N_CHUNKS = B_PER_W // CHUNK
NBUF = 4

_mesh = plsc.VectorSubcoreMesh(core_axis_name="c", subcore_axis_name="s")


@functools.partial(
    pl.kernel,
    out_type=jax.ShapeDtypeStruct((TOTAL, EMBED_DIM), jnp.float32),
    mesh=_mesh,
    scratch_types=[
        pltpu.VMEM((B_PER_W,), jnp.int32),
        pltpu.VMEM((NBUF, CHUNK, EMBED_DIM), jnp.float32),
        [pltpu.SemaphoreType.DMA] * NBUF,
        [pltpu.SemaphoreType.DMA] * NBUF,
    ],
    compiler_params=pltpu.CompilerParams(use_tc_tiling_on_sc=False),
)
def _embed_gather(idx_hbm, table_hbm, out_hbm, idx_v, rows_v, gsems, ssems):
    wid = lax.axis_index("s") * NC + lax.axis_index("c")
    base = wid * B_PER_W
    pltpu.sync_copy(idx_hbm.at[pl.ds(base, B_PER_W)], idx_v)

    def start_gather(c):
        b = c % NBUF
        return pltpu.async_copy(
            table_hbm.at[idx_v.at[pl.ds(c * CHUNK, CHUNK)]],
            rows_v.at[b],
            gsems[b],
        )

    def start_store(c):
        b = c % NBUF
        return pltpu.async_copy(
            rows_v.at[b],
            out_hbm.at[pl.ds(base + c * CHUNK, CHUNK)],
            ssems[b],
        )

    gathers = [None] * N_CHUNKS
    stores = [None] * N_CHUNKS
    for c in range(min(NBUF - 1, N_CHUNKS)):
        gathers[c] = start_gather(c)
    for c in range(N_CHUNKS):
        if c > 0:
            stores[c - 1].wait()
        g = c + NBUF - 1
        if g < N_CHUNKS:
            gathers[g] = start_gather(g)
        gathers[c].wait()
        stores[c] = start_store(c)
    stores[N_CHUNKS - 1].wait()


def kernel(x, table):
    flat = x.reshape(TOTAL).astype(jnp.int32) * 4
    tpad = jnp.pad(table, ((0, 0), (0, 128 - EMBED_DIM)))
    t4 = tpad.reshape(4 * VOCAB_ROWS, EMBED_DIM)
    out = _embed_gather(flat, t4)
    return out.reshape(BATCH, NUM_FIELDS, EMBED_DIM)
